# baseline (speedup 1.0000x reference)
import jax
import jax.numpy as jnp
from jax import lax
from jax.experimental import pallas as pl
from jax.experimental.pallas import tpu as pltpu


def _pool_body(x_ref, out_ref, *, hw):
    s1 = jnp.sum(x_ref[0], axis=2)
    out_ref[0, 0, :] = jnp.sum(s1, axis=1) / jnp.float32(hw)


def _rank_body(r_ref, rt_ref, idx_ref, *, bsz, c, k):
    jrow = jax.lax.broadcasted_iota(jnp.int32, (c, c), 0)
    icol = jax.lax.broadcasted_iota(jnp.int32, (c, c), 1)
    prow = jax.lax.broadcasted_iota(jnp.int32, (k, c), 0)
    ccol = jax.lax.broadcasted_iota(jnp.int32, (k, c), 1)
    for b in range(bsz):
        r_lane = jnp.broadcast_to(r_ref[b:b + 1, :], (c, c))
        r_sub = jnp.broadcast_to(rt_ref[:, b:b + 1], (c, c))
        m = (r_sub > r_lane) | ((r_sub == r_lane) & (jrow < icol))
        rank = jnp.sum(m.astype(jnp.int32), axis=0, keepdims=True)
        sel = jnp.broadcast_to(rank, (k, c)) == prow
        idx_ref[:, b:b + 1] = jnp.sum(jnp.where(sel, ccol, 0), axis=1,
                                      keepdims=True)


def _gather_body(idx_ref, x_hbm, out_hbm, buf, isem, osem,
                 *, bsz, k, gw, h, w):
    nch = k // gw
    total = bsz * nch

    def issue_in(t, p):
        b = t // nch
        j0 = lax.rem(t, nch) * gw
        for i in range(gw):
            pltpu.make_async_copy(
                x_hbm.at[pl.ds(b, 1), pl.ds(idx_ref[b, j0 + i], 1)],
                buf.at[pl.ds(p, 1), pl.ds(i, 1)],
                isem.at[p]).start()

    def wait_in(p):
        for _ in range(gw):
            pltpu.make_async_copy(
                x_hbm.at[pl.ds(0, 1), pl.ds(0, 1)],
                buf.at[pl.ds(p, 1), pl.ds(0, 1)],
                isem.at[p]).wait()

    def issue_out(t, q):
        b = t // nch
        j0 = lax.rem(t, nch) * gw
        pltpu.make_async_copy(
            buf.at[pl.ds(q, 1)],
            out_hbm.at[pl.ds(b, 1), pl.ds(j0, gw)],
            osem.at[q]).start()

    def wait_out(q):
        pltpu.make_async_copy(
            buf.at[pl.ds(q, 1)],
            out_hbm.at[pl.ds(0, 1), pl.ds(0, gw)],
            osem.at[q]).wait()

    issue_in(0, 0)

    def step(t, carry):
        p = lax.rem(t, 2)
        q = lax.rem(t - 1, 2)

        @pl.when(t >= 2)
        def _():
            wait_out(p)

        @pl.when(t < total)
        def _():
            issue_in(t, p)

        wait_in(q)
        issue_out(t - 1, q)
        return carry

    lax.fori_loop(1, total + 1, step, 0)
    wait_out(lax.rem(total - 1, 2))


def kernel(x, conv_w):
    B, C, H, W = x.shape
    HW = H * W
    k = int(C * 0.5)
    CB = 16

    pooled3 = pl.pallas_call(
        lambda xr, orf: _pool_body(xr, orf, hw=HW),
        grid=(B, C // CB),
        in_specs=[pl.BlockSpec((1, CB, H, W), lambda b, i: (b, i, 0, 0))],
        out_specs=pl.BlockSpec((1, 1, CB), lambda b, i: (b * (C // CB) + i, 0, 0)),
        out_shape=jax.ShapeDtypeStruct((B * C // CB, 1, CB), jnp.float32),
    )(x)
    pooled = pooled3.reshape(B, C)

    padded = jnp.pad(pooled, ((0, 0), (1, 1)))
    conv = (conv_w[0] * padded[:, :-2] + conv_w[1] * padded[:, 1:-1]
            + conv_w[2] * padded[:, 2:])
    r = jax.nn.sigmoid(conv)

    idx_t = pl.pallas_call(
        lambda rr, rt, ir: _rank_body(rr, rt, ir, bsz=B, c=C, k=k),
        out_shape=jax.ShapeDtypeStruct((k, B), jnp.int32),
    )(r, r.T)
    idx = idx_t.T

    GW = 16
    grid_spec = pltpu.PrefetchScalarGridSpec(
        num_scalar_prefetch=1,
        grid=(1,),
        in_specs=[pl.BlockSpec(memory_space=pltpu.MemorySpace.HBM)],
        out_specs=pl.BlockSpec(memory_space=pltpu.MemorySpace.HBM),
        scratch_shapes=[
            pltpu.VMEM((2, GW, H, W), jnp.float32),
            pltpu.SemaphoreType.DMA((2,)),
            pltpu.SemaphoreType.DMA((2,)),
        ],
    )
    out = pl.pallas_call(
        lambda ir, xr, orf, buf, isem, osem: _gather_body(
            ir, xr, orf, buf, isem, osem, bsz=B, k=k, gw=GW, h=H, w=W),
        grid_spec=grid_spec,
        out_shape=jax.ShapeDtypeStruct((B, k, H, W), jnp.float32),
    )(idx, x)
    return out


# speedup vs baseline: 1.0848x; 1.0848x over previous
import jax
import jax.numpy as jnp
from jax import lax
from jax.experimental import pallas as pl
from jax.experimental.pallas import tpu as pltpu


def _pool_body(x_ref, out_ref, *, hw):
    s1 = jnp.sum(x_ref[0], axis=2)
    out_ref[0, 0, :] = jnp.sum(s1, axis=1) / jnp.float32(hw)


def _rank_body(r_ref, rt_ref, idx_ref, *, bsz, c, k):
    jrow = jax.lax.broadcasted_iota(jnp.int32, (c, c), 0)
    icol = jax.lax.broadcasted_iota(jnp.int32, (c, c), 1)
    prow = jax.lax.broadcasted_iota(jnp.int32, (k, c), 0)
    ccol = jax.lax.broadcasted_iota(jnp.int32, (k, c), 1)
    for b in range(bsz):
        r_lane = jnp.broadcast_to(r_ref[b:b + 1, :], (c, c))
        r_sub = jnp.broadcast_to(rt_ref[:, b:b + 1], (c, c))
        m = (r_sub > r_lane) | ((r_sub == r_lane) & (jrow < icol))
        rank = jnp.sum(m.astype(jnp.int32), axis=0, keepdims=True)
        sel = jnp.broadcast_to(rank, (k, c)) == prow
        idx_ref[:, b:b + 1] = jnp.sum(jnp.where(sel, ccol, 0), axis=1,
                                      keepdims=True)


def _gather_body(idx_ref, x_hbm, out_hbm, buf, isem, osem,
                 *, bsz, k, gw, h, w):
    NB = 3
    nch = k // gw
    total = bsz * nch

    def issue_in(t, p):
        b = t // nch
        j0 = lax.rem(t, nch) * gw
        for i in range(gw):
            pltpu.make_async_copy(
                x_hbm.at[pl.ds(b, 1), pl.ds(idx_ref[b, j0 + i], 1)],
                buf.at[pl.ds(p, 1), pl.ds(i, 1)],
                isem.at[p]).start()

    def wait_in(p):
        for _ in range(gw):
            pltpu.make_async_copy(
                x_hbm.at[pl.ds(0, 1), pl.ds(0, 1)],
                buf.at[pl.ds(p, 1), pl.ds(0, 1)],
                isem.at[p]).wait()

    def issue_out(t, q):
        b = t // nch
        j0 = lax.rem(t, nch) * gw
        pltpu.make_async_copy(
            buf.at[pl.ds(q, 1)],
            out_hbm.at[pl.ds(b, 1), pl.ds(j0, gw)],
            osem.at[q]).start()

    def wait_out(q):
        pltpu.make_async_copy(
            buf.at[pl.ds(q, 1)],
            out_hbm.at[pl.ds(0, 1), pl.ds(0, gw)],
            osem.at[q]).wait()

    issue_in(0, 0)

    def step(t, carry):
        p = lax.rem(t, NB)
        q = lax.rem(t - 1, NB)

        @pl.when(t >= NB)
        def _():
            wait_out(p)

        @pl.when(t < total)
        def _():
            issue_in(t, p)

        wait_in(q)
        issue_out(t - 1, q)
        return carry

    lax.fori_loop(1, total + 1, step, 0)
    for cnk in range(total - NB + 1, total):
        wait_out(cnk % NB)


def kernel(x, conv_w):
    B, C, H, W = x.shape
    HW = H * W
    k = int(C * 0.5)
    CB = 32

    pooled3 = pl.pallas_call(
        lambda xr, orf: _pool_body(xr, orf, hw=HW),
        grid=(B, C // CB),
        in_specs=[pl.BlockSpec((1, CB, H, W), lambda b, i: (b, i, 0, 0))],
        out_specs=pl.BlockSpec((1, 1, CB), lambda b, i: (b * (C // CB) + i, 0, 0)),
        out_shape=jax.ShapeDtypeStruct((B * C // CB, 1, CB), jnp.float32),
    )(x)
    pooled = pooled3.reshape(B, C)

    padded = jnp.pad(pooled, ((0, 0), (1, 1)))
    conv = (conv_w[0] * padded[:, :-2] + conv_w[1] * padded[:, 1:-1]
            + conv_w[2] * padded[:, 2:])
    r = jax.nn.sigmoid(conv)

    idx_t = pl.pallas_call(
        lambda rr, rt, ir: _rank_body(rr, rt, ir, bsz=B, c=C, k=k),
        out_shape=jax.ShapeDtypeStruct((k, B), jnp.int32),
    )(r, r.T)
    idx = idx_t.T

    GW = 16
    grid_spec = pltpu.PrefetchScalarGridSpec(
        num_scalar_prefetch=1,
        grid=(1,),
        in_specs=[pl.BlockSpec(memory_space=pltpu.MemorySpace.HBM)],
        out_specs=pl.BlockSpec(memory_space=pltpu.MemorySpace.HBM),
        scratch_shapes=[
            pltpu.VMEM((3, GW, H, W), jnp.float32),
            pltpu.SemaphoreType.DMA((3,)),
            pltpu.SemaphoreType.DMA((3,)),
        ],
    )
    out = pl.pallas_call(
        lambda ir, xr, orf, buf, isem, osem: _gather_body(
            ir, xr, orf, buf, isem, osem, bsz=B, k=k, gw=GW, h=H, w=W),
        grid_spec=grid_spec,
        out_shape=jax.ShapeDtypeStruct((B, k, H, W), jnp.float32),
    )(idx, x)
    return out


# speedup vs baseline: 1.0854x; 1.0006x over previous
import jax
import jax.numpy as jnp
from jax import lax
from jax.experimental import pallas as pl
from jax.experimental.pallas import tpu as pltpu


def _pool_body(x_ref, out_ref, *, hw):
    s1 = jnp.sum(x_ref[0], axis=2)
    out_ref[0, 0, :] = jnp.sum(s1, axis=1) / jnp.float32(hw)


def _rank_body(r_ref, rt_ref, idx_ref, *, bsz, c, k):
    jrow = jax.lax.broadcasted_iota(jnp.int32, (c, c), 0)
    icol = jax.lax.broadcasted_iota(jnp.int32, (c, c), 1)
    prow = jax.lax.broadcasted_iota(jnp.int32, (k, c), 0)
    ccol = jax.lax.broadcasted_iota(jnp.int32, (k, c), 1)
    for b in range(bsz):
        r_lane = jnp.broadcast_to(r_ref[b:b + 1, :], (c, c))
        r_sub = jnp.broadcast_to(rt_ref[:, b:b + 1], (c, c))
        m = (r_sub > r_lane) | ((r_sub == r_lane) & (jrow < icol))
        rank = jnp.sum(m.astype(jnp.int32), axis=0, keepdims=True)
        sel = jnp.broadcast_to(rank, (k, c)) == prow
        idx_ref[:, b:b + 1] = jnp.sum(jnp.where(sel, ccol, 0), axis=1,
                                      keepdims=True)


def _gather_body(idx_ref, x_hbm, out_hbm, buf, isem, osem,
                 *, bsz, k, gw, h, w):
    NB = 3
    nch = k // gw
    total = bsz * nch

    def issue_in(t, p, pin):
        b = t // nch
        j0 = lax.rem(t, nch) * gw
        for i in range(gw):
            pltpu.make_async_copy(
                x_hbm.at[pl.ds(b, 1), pl.ds(idx_ref[b, j0 + i], 1)],
                buf.at[pl.ds(p, 1), pl.ds(i, 1)],
                isem.at[pin, i]).start()

    def wait_in(pin):
        for i in range(gw):
            pltpu.make_async_copy(
                x_hbm.at[pl.ds(0, 1), pl.ds(0, 1)],
                buf.at[pl.ds(0, 1), pl.ds(0, 1)],
                isem.at[pin, i]).wait()

    def issue_out(t, q):
        b = t // nch
        j0 = lax.rem(t, nch) * gw
        pltpu.make_async_copy(
            buf.at[pl.ds(q, 1)],
            out_hbm.at[pl.ds(b, 1), pl.ds(j0, gw)],
            osem.at[q]).start()

    def wait_out(q):
        pltpu.make_async_copy(
            buf.at[pl.ds(q, 1)],
            out_hbm.at[pl.ds(0, 1), pl.ds(0, gw)],
            osem.at[q]).wait()

    issue_in(0, 0, 0)

    def step(t, carry):
        p = lax.rem(t, NB)
        q = lax.rem(t - 1, NB)
        pin = lax.rem(t, 2)
        qin = lax.rem(t - 1, 2)

        @pl.when(t >= NB)
        def _():
            wait_out(p)

        @pl.when(t < total)
        def _():
            issue_in(t, p, pin)

        wait_in(qin)
        issue_out(t - 1, q)
        return carry

    lax.fori_loop(1, total + 1, step, 0)
    for cnk in range(total - NB + 1, total):
        wait_out(cnk % NB)


def kernel(x, conv_w):
    B, C, H, W = x.shape
    HW = H * W
    k = int(C * 0.5)
    CB = 64

    pooled3 = pl.pallas_call(
        lambda xr, orf: _pool_body(xr, orf, hw=HW),
        grid=(B, C // CB),
        in_specs=[pl.BlockSpec((1, CB, H, W), lambda b, i: (b, i, 0, 0))],
        out_specs=pl.BlockSpec((1, 1, CB), lambda b, i: (b * (C // CB) + i, 0, 0)),
        out_shape=jax.ShapeDtypeStruct((B * C // CB, 1, CB), jnp.float32),
    )(x)
    pooled = pooled3.reshape(B, C)

    padded = jnp.pad(pooled, ((0, 0), (1, 1)))
    conv = (conv_w[0] * padded[:, :-2] + conv_w[1] * padded[:, 1:-1]
            + conv_w[2] * padded[:, 2:])
    r = jax.nn.sigmoid(conv)

    idx_t = pl.pallas_call(
        lambda rr, rt, ir: _rank_body(rr, rt, ir, bsz=B, c=C, k=k),
        out_shape=jax.ShapeDtypeStruct((k, B), jnp.int32),
    )(r, r.T)
    idx = idx_t.T

    GW = 16
    grid_spec = pltpu.PrefetchScalarGridSpec(
        num_scalar_prefetch=1,
        grid=(1,),
        in_specs=[pl.BlockSpec(memory_space=pltpu.MemorySpace.HBM)],
        out_specs=pl.BlockSpec(memory_space=pltpu.MemorySpace.HBM),
        scratch_shapes=[
            pltpu.VMEM((3, GW, H, W), jnp.float32),
            pltpu.SemaphoreType.DMA((2, GW)),
            pltpu.SemaphoreType.DMA((3,)),
        ],
    )
    out = pl.pallas_call(
        lambda ir, xr, orf, buf, isem, osem: _gather_body(
            ir, xr, orf, buf, isem, osem, bsz=B, k=k, gw=GW, h=H, w=W),
        grid_spec=grid_spec,
        out_shape=jax.ShapeDtypeStruct((B, k, H, W), jnp.float32),
    )(idx, x)
    return out
